# baseline (speedup 1.0000x reference)
import functools

import jax
import jax.numpy as jnp
from jax.experimental import pallas as pl
from jax.experimental.pallas import tpu as pltpu

PAIRS_PER_BLOCK = 4

_NPG = 64
_EPG = 128
_PAIR_N = 2 * _NPG
_PAIR_E = 2 * _EPG
_N_PROP = 3
_S_ITERS = 10
_INV_TEMP = 10.0


def _mm(a, b):
    return jax.lax.dot_general(a, b, (((1,), (0,)), ((), ())),
                               preferred_element_type=jnp.float32)


def _mm_nt(a, b):
    return jax.lax.dot_general(a, b, (((1,), (1,)), ((), ())),
                               preferred_element_type=jnp.float32)


def _mm_tn(a, b):
    return jax.lax.dot_general(a, b, (((0,), (0,)), ((), ())),
                               preferred_element_type=jnp.float32)


def _lse(x, axis):
    m = jnp.max(x, axis=axis, keepdims=True)
    return m + jnp.log(jnp.sum(jnp.exp(x - m), axis=axis, keepdims=True))


def _lse_rows(x, pairs, n):
    x3 = x.reshape(pairs, n, x.shape[1])
    m = jnp.max(x3, axis=1, keepdims=True)
    r = m + jnp.log(jnp.sum(jnp.exp(x3 - m), axis=1, keepdims=True))
    return jnp.broadcast_to(r, x3.shape).reshape(x.shape)


def _sinkhorn2(sim_n, sim_e, pairs):
    la_n = sim_n * _INV_TEMP
    la_e = sim_e * _INV_TEMP
    for _ in range(_S_ITERS):
        la_n = la_n - _lse(la_n, 1)
        la_e = la_e - _lse(la_e, 1)
        la_n = la_n - _lse_rows(la_n, pairs, _NPG)
        la_e = la_e - _lse_rows(la_e, pairs, _EPG)
    return jnp.exp(la_n), jnp.exp(la_e)


def _block_kernel(nf_ref, ef_ref, fic_ref, tic_ref,
                  wen_ref, ben_ref, wee_ref, bee_ref,
                  wsd_ref, wer_ref, beb_ref, wup_ref, bup_ref,
                  wn1_ref, bn1_ref, wn2_ref, bn2_ref,
                  we1_ref, be1_ref, we2_ref, be2_ref,
                  out_ref, *, pairs):
    f32 = jnp.float32

    x = _mm(nf_ref[...], wen_ref[...]) + ben_ref[...]
    eenc = _mm(ef_ref[...], wee_ref[...]) + bee_ref[...]
    eb = _mm(eenc, wer_ref[...]) + beb_ref[...]

    fic = jnp.bitwise_and(fic_ref[0], _PAIR_N - 1)
    tic = jnp.bitwise_and(tic_ref[0], _PAIR_N - 1)

    lane = jax.lax.broadcasted_iota(jnp.int32, (_PAIR_E, _PAIR_N), 1)

    ohf, oht = [], []
    for j in range(pairs):
        e0 = j * _PAIR_E
        fc = fic[e0:e0 + _PAIR_E, :]
        tc = tic[e0:e0 + _PAIR_E, :]
        ohf.append(jnp.where(fc == lane, 1.0, 0.0).astype(f32))
        oht.append(jnp.where(tc == lane, 1.0, 0.0).astype(f32))

    def messages(xcur):
        sds = []
        for j in range(pairs):
            xj = xcur[j * _PAIR_N:(j + 1) * _PAIR_N, :]
            src = _mm(ohf[j], xj)
            dst = _mm(oht[j], xj)
            sds.append(jnp.concatenate([src, dst], axis=1))
        sd = jnp.concatenate(sds, axis=0)
        return jax.nn.relu(_mm(sd, wsd_ref[...]) + eb)

    for _ in range(_N_PROP):
        msg = messages(x)
        aggs = []
        for j in range(pairs):
            mj = msg[j * _PAIR_E:(j + 1) * _PAIR_E, :]
            fwd = mj[:, :128]
            rev = mj[:, 128:]
            aggs.append(_mm_tn(oht[j], fwd) + _mm_tn(ohf[j], rev))
        agg = jnp.concatenate(aggs, axis=0)
        xin = jnp.concatenate([x, agg], axis=1)
        x = _mm(xin, wup_ref[...]) + bup_ref[...]

    msg = messages(x)
    emsg = msg[:, :128] + msg[:, 128:]

    hn = _mm(jax.nn.relu(_mm(x, wn1_ref[...]) + bn1_ref[...]),
             wn2_ref[...]) + bn2_ref[...]
    he = _mm(jax.nn.relu(_mm(emsg, we1_ref[...]) + be1_ref[...]),
             we2_ref[...]) + be2_ref[...]

    sims_n, sims_e = [], []
    for j in range(pairs):
        n0 = j * _PAIR_N
        tq = hn[n0:n0 + _NPG, :]
        tc = hn[n0 + _NPG:n0 + _PAIR_N, :]
        sims_n.append(_mm_nt(tq, tc))
        e0 = j * _PAIR_E
        tqe = he[e0:e0 + _EPG, :]
        tce = he[e0 + _EPG:e0 + _PAIR_E, :]
        sims_e.append(_mm_nt(tqe, tce))

    pn_all, pe_all = _sinkhorn2(jnp.concatenate(sims_n, axis=0),
                                jnp.concatenate(sims_e, axis=0), pairs)

    scores = []
    for j in range(pairs):
        n0 = j * _PAIR_N
        tq = hn[n0:n0 + _NPG, :]
        tc = hn[n0 + _NPG:n0 + _PAIR_N, :]
        pn = pn_all[j * _NPG:(j + 1) * _NPG, :]
        s1 = jnp.sum(jax.nn.relu(tq - _mm(pn, tc)),
                     axis=(0, 1), keepdims=True)
        e0 = j * _PAIR_E
        tqe = he[e0:e0 + _EPG, :]
        tce = he[e0 + _EPG:e0 + _PAIR_E, :]
        pe = pe_all[j * _EPG:(j + 1) * _EPG, :]
        s2 = jnp.sum(jax.nn.relu(tqe - _mm(pe, tce)),
                     axis=(0, 1), keepdims=True)
        scores.append(-(s1 + s2))
    out_ref[...] = jnp.concatenate(scores, axis=1).reshape(1, 1, pairs)


def kernel(node_features, edge_features, from_idx, to_idx,
           W_enc_n, b_enc_n, W_enc_e, b_enc_e,
           W_msg, b_msg, W_rmsg, b_rmsg, W_upd, b_upd,
           Wn1, bn1, Wn2, bn2, We1, be1, We2, be2):
    f32 = jnp.float32
    n_total = node_features.shape[0]
    n_pairs = n_total // _PAIR_N
    e_total = n_pairs * _PAIR_E
    p = PAIRS_PER_BLOCK
    while n_pairs % p:
        p //= 2
    nb = n_pairs // p

    wm_s, wm_d, wm_e = W_msg[:64], W_msg[64:128], W_msg[128:]
    wr_d, wr_s, wr_e = W_rmsg[:64], W_rmsg[64:128], W_rmsg[128:]
    w_sd = jnp.concatenate([
        jnp.concatenate([wm_s, wr_s], axis=1),
        jnp.concatenate([wm_d, wr_d], axis=1),
    ], axis=0)
    w_er = jnp.concatenate([wm_e, wr_e], axis=1)
    b_eb = jnp.concatenate([b_msg, b_rmsg]).reshape(1, 256)

    fic = from_idx.reshape(nb, p * _PAIR_E, 1)
    tic = to_idx.reshape(nb, p * _PAIR_E, 1)

    def full(arr):
        a = jnp.asarray(arr, f32)
        if a.ndim == 1:
            a = a.reshape(1, -1)
        return a, pl.BlockSpec(a.shape, lambda i: (0,) * a.ndim)

    weights = [W_enc_n, b_enc_n, W_enc_e, b_enc_e,
               w_sd, w_er, b_eb, W_upd, b_upd,
               Wn1, bn1, Wn2, bn2, We1, be1, We2, be2]
    warrs, wspecs = zip(*[full(w) for w in weights])

    out = pl.pallas_call(
        functools.partial(_block_kernel, pairs=p),
        grid=(nb,),
        in_specs=[
            pl.BlockSpec((p * _PAIR_N, node_features.shape[1]),
                         lambda i: (i, 0)),
            pl.BlockSpec((p * _PAIR_E, edge_features.shape[1]),
                         lambda i: (i, 0)),
            pl.BlockSpec((1, p * _PAIR_E, 1), lambda i: (i, 0, 0)),
            pl.BlockSpec((1, p * _PAIR_E, 1), lambda i: (i, 0, 0)),
            *wspecs,
        ],
        out_specs=pl.BlockSpec((1, 1, p), lambda i: (i, 0, 0)),
        out_shape=jax.ShapeDtypeStruct((nb, 1, p), f32),
        compiler_params=pltpu.CompilerParams(
            dimension_semantics=(pltpu.GridDimensionSemantics.ARBITRARY,)),
    )(node_features, edge_features, fic, tic, *warrs)
    return out.reshape(n_pairs)


# speedup vs baseline: 1.3010x; 1.3010x over previous
import functools

import jax
import jax.numpy as jnp
from jax.experimental import pallas as pl
from jax.experimental.pallas import tpu as pltpu

PAIRS_PER_BLOCK = 8

_NPG = 64
_EPG = 128
_PAIR_N = 2 * _NPG
_PAIR_E = 2 * _EPG
_N_PROP = 3
_S_ITERS = 10
_INV_TEMP = 10.0


def _mm(a, b):
    return jax.lax.dot_general(a, b, (((1,), (0,)), ((), ())),
                               preferred_element_type=jnp.float32)


def _mm_nt(a, b):
    return jax.lax.dot_general(a, b, (((1,), (1,)), ((), ())),
                               preferred_element_type=jnp.float32)


def _mm_tn(a, b):
    return jax.lax.dot_general(a, b, (((0,), (0,)), ((), ())),
                               preferred_element_type=jnp.float32)


def _lse(x, axis):
    m = jnp.max(x, axis=axis, keepdims=True)
    return m + jnp.log(jnp.sum(jnp.exp(x - m), axis=axis, keepdims=True))


def _lse_rows(x, pairs, n):
    x3 = x.reshape(pairs, n, x.shape[1])
    m = jnp.max(x3, axis=1, keepdims=True)
    r = m + jnp.log(jnp.sum(jnp.exp(x3 - m), axis=1, keepdims=True))
    return jnp.broadcast_to(r, x3.shape).reshape(x.shape)


def _sinkhorn2(sim_n, sim_e, pairs):
    la_n = sim_n * _INV_TEMP
    la_e = sim_e * _INV_TEMP
    for _ in range(_S_ITERS):
        la_n = la_n - _lse(la_n, 1)
        la_e = la_e - _lse(la_e, 1)
        la_n = la_n - _lse_rows(la_n, pairs, _NPG)
        la_e = la_e - _lse_rows(la_e, pairs, _EPG)
    return jnp.exp(la_n), jnp.exp(la_e)


def _block_kernel(nf_ref, ef_ref, fic_ref, tic_ref, fir_ref, tir_ref,
                  wen_ref, ben_ref, wee_ref, bee_ref,
                  wsd_ref, wer_ref, beb_ref, wup_ref, bup_ref,
                  wn1_ref, bn1_ref, wn2_ref, bn2_ref,
                  we1_ref, be1_ref, we2_ref, be2_ref,
                  out_ref, *, pairs):
    f32 = jnp.float32

    x = _mm(nf_ref[...], wen_ref[...]) + ben_ref[...]
    eenc = _mm(ef_ref[...], wee_ref[...]) + bee_ref[...]
    eb = _mm(eenc, wer_ref[...]) + beb_ref[...]

    fic = jnp.bitwise_and(fic_ref[0], _PAIR_N - 1)
    tic = jnp.bitwise_and(tic_ref[0], _PAIR_N - 1)
    fir = jnp.bitwise_and(fir_ref[0], _PAIR_N - 1)
    tir = jnp.bitwise_and(tir_ref[0], _PAIR_N - 1)

    lane = jax.lax.broadcasted_iota(jnp.int32, (_PAIR_E, _PAIR_N), 1)
    sub = jax.lax.broadcasted_iota(jnp.int32, (_PAIR_N, _PAIR_E), 0)

    ohf, oht, ohtf, ohtt = [], [], [], []
    for j in range(pairs):
        e0 = j * _PAIR_E
        fc = fic[e0:e0 + _PAIR_E, :]
        tc = tic[e0:e0 + _PAIR_E, :]
        fr = fir[:, e0:e0 + _PAIR_E]
        tr = tir[:, e0:e0 + _PAIR_E]
        ohf.append(jnp.where(fc == lane, 1.0, 0.0).astype(f32))
        oht.append(jnp.where(tc == lane, 1.0, 0.0).astype(f32))
        ohtf.append(jnp.where(fr == sub, 1.0, 0.0).astype(f32))
        ohtt.append(jnp.where(tr == sub, 1.0, 0.0).astype(f32))

    def messages(xcur):
        sds = []
        for j in range(pairs):
            xj = xcur[j * _PAIR_N:(j + 1) * _PAIR_N, :]
            src = _mm(ohf[j], xj)
            dst = _mm(oht[j], xj)
            sds.append(jnp.concatenate([src, dst], axis=1))
        sd = jnp.concatenate(sds, axis=0)
        return jax.nn.relu(_mm(sd, wsd_ref[...]) + eb)

    for _ in range(_N_PROP):
        msg = messages(x)
        aggs = []
        for j in range(pairs):
            mj = msg[j * _PAIR_E:(j + 1) * _PAIR_E, :]
            fwd = mj[:, :128]
            rev = mj[:, 128:]
            aggs.append(_mm(ohtt[j], fwd) + _mm(ohtf[j], rev))
        agg = jnp.concatenate(aggs, axis=0)
        xin = jnp.concatenate([x, agg], axis=1)
        x = _mm(xin, wup_ref[...]) + bup_ref[...]

    msg = messages(x)
    emsg = msg[:, :128] + msg[:, 128:]

    hn = _mm(jax.nn.relu(_mm(x, wn1_ref[...]) + bn1_ref[...]),
             wn2_ref[...]) + bn2_ref[...]
    he = _mm(jax.nn.relu(_mm(emsg, we1_ref[...]) + be1_ref[...]),
             we2_ref[...]) + be2_ref[...]

    sims_n, sims_e = [], []
    for j in range(pairs):
        n0 = j * _PAIR_N
        tq = hn[n0:n0 + _NPG, :]
        tc = hn[n0 + _NPG:n0 + _PAIR_N, :]
        sims_n.append(_mm_nt(tq, tc))
        e0 = j * _PAIR_E
        tqe = he[e0:e0 + _EPG, :]
        tce = he[e0 + _EPG:e0 + _PAIR_E, :]
        sims_e.append(_mm_nt(tqe, tce))

    pn_all, pe_all = _sinkhorn2(jnp.concatenate(sims_n, axis=0),
                                jnp.concatenate(sims_e, axis=0), pairs)

    scores = []
    for j in range(pairs):
        n0 = j * _PAIR_N
        tq = hn[n0:n0 + _NPG, :]
        tc = hn[n0 + _NPG:n0 + _PAIR_N, :]
        pn = pn_all[j * _NPG:(j + 1) * _NPG, :]
        s1 = jnp.sum(jax.nn.relu(tq - _mm(pn, tc)),
                     axis=(0, 1), keepdims=True)
        e0 = j * _PAIR_E
        tqe = he[e0:e0 + _EPG, :]
        tce = he[e0 + _EPG:e0 + _PAIR_E, :]
        pe = pe_all[j * _EPG:(j + 1) * _EPG, :]
        s2 = jnp.sum(jax.nn.relu(tqe - _mm(pe, tce)),
                     axis=(0, 1), keepdims=True)
        scores.append(-(s1 + s2))
    out_ref[...] = jnp.concatenate(scores, axis=1).reshape(1, 1, pairs)


def kernel(node_features, edge_features, from_idx, to_idx,
           W_enc_n, b_enc_n, W_enc_e, b_enc_e,
           W_msg, b_msg, W_rmsg, b_rmsg, W_upd, b_upd,
           Wn1, bn1, Wn2, bn2, We1, be1, We2, be2):
    f32 = jnp.float32
    n_total = node_features.shape[0]
    n_pairs = n_total // _PAIR_N
    e_total = n_pairs * _PAIR_E
    p = PAIRS_PER_BLOCK
    while n_pairs % p:
        p //= 2
    nb = n_pairs // p

    wm_s, wm_d, wm_e = W_msg[:64], W_msg[64:128], W_msg[128:]
    wr_d, wr_s, wr_e = W_rmsg[:64], W_rmsg[64:128], W_rmsg[128:]
    w_sd = jnp.concatenate([
        jnp.concatenate([wm_s, wr_s], axis=1),
        jnp.concatenate([wm_d, wr_d], axis=1),
    ], axis=0)
    w_er = jnp.concatenate([wm_e, wr_e], axis=1)
    b_eb = jnp.concatenate([b_msg, b_rmsg]).reshape(1, 256)

    fic = from_idx.reshape(nb, p * _PAIR_E, 1)
    tic = to_idx.reshape(nb, p * _PAIR_E, 1)
    fir = from_idx.reshape(nb, 1, p * _PAIR_E)
    tir = to_idx.reshape(nb, 1, p * _PAIR_E)

    def full(arr):
        a = jnp.asarray(arr, f32)
        if a.ndim == 1:
            a = a.reshape(1, -1)
        return a, pl.BlockSpec(a.shape, lambda i: (0,) * a.ndim)

    weights = [W_enc_n, b_enc_n, W_enc_e, b_enc_e,
               w_sd, w_er, b_eb, W_upd, b_upd,
               Wn1, bn1, Wn2, bn2, We1, be1, We2, be2]
    warrs, wspecs = zip(*[full(w) for w in weights])

    out = pl.pallas_call(
        functools.partial(_block_kernel, pairs=p),
        grid=(nb,),
        in_specs=[
            pl.BlockSpec((p * _PAIR_N, node_features.shape[1]),
                         lambda i: (i, 0)),
            pl.BlockSpec((p * _PAIR_E, edge_features.shape[1]),
                         lambda i: (i, 0)),
            pl.BlockSpec((1, p * _PAIR_E, 1), lambda i: (i, 0, 0)),
            pl.BlockSpec((1, p * _PAIR_E, 1), lambda i: (i, 0, 0)),
            pl.BlockSpec((1, 1, p * _PAIR_E), lambda i: (i, 0, 0)),
            pl.BlockSpec((1, 1, p * _PAIR_E), lambda i: (i, 0, 0)),
            *wspecs,
        ],
        out_specs=pl.BlockSpec((1, 1, p), lambda i: (i, 0, 0)),
        out_shape=jax.ShapeDtypeStruct((nb, 1, p), f32),
        compiler_params=pltpu.CompilerParams(
            dimension_semantics=(pltpu.GridDimensionSemantics.ARBITRARY,)),
    )(node_features, edge_features, fic, tic, fir, tir, *warrs)
    return out.reshape(n_pairs)


# speedup vs baseline: 1.3667x; 1.0505x over previous
import functools

import jax
import jax.numpy as jnp
from jax.experimental import pallas as pl
from jax.experimental.pallas import tpu as pltpu

PAIRS_PER_BLOCK = 16

_NPG = 64
_EPG = 128
_PAIR_N = 2 * _NPG
_PAIR_E = 2 * _EPG
_N_PROP = 3
_S_ITERS = 10
_INV_TEMP = 10.0


def _mm(a, b):
    return jax.lax.dot_general(a, b, (((1,), (0,)), ((), ())),
                               preferred_element_type=jnp.float32)


def _mm_nt(a, b):
    return jax.lax.dot_general(a, b, (((1,), (1,)), ((), ())),
                               preferred_element_type=jnp.float32)


def _mm_tn(a, b):
    return jax.lax.dot_general(a, b, (((0,), (0,)), ((), ())),
                               preferred_element_type=jnp.float32)


def _lse(x, axis):
    m = jnp.max(x, axis=axis, keepdims=True)
    return m + jnp.log(jnp.sum(jnp.exp(x - m), axis=axis, keepdims=True))


def _lse_rows(x, pairs, n):
    x3 = x.reshape(pairs, n, x.shape[1])
    m = jnp.max(x3, axis=1, keepdims=True)
    r = m + jnp.log(jnp.sum(jnp.exp(x3 - m), axis=1, keepdims=True))
    return jnp.broadcast_to(r, x3.shape).reshape(x.shape)


def _sinkhorn2(sim_n, sim_e, pairs):
    la_n = sim_n * _INV_TEMP
    la_e = sim_e * _INV_TEMP
    for _ in range(_S_ITERS):
        la_n = la_n - _lse(la_n, 1)
        la_e = la_e - _lse(la_e, 1)
        la_n = la_n - _lse_rows(la_n, pairs, _NPG)
        la_e = la_e - _lse_rows(la_e, pairs, _EPG)
    return jnp.exp(la_n), jnp.exp(la_e)


def _block_kernel(nf_ref, ef_ref, fic_ref, tic_ref, fir_ref, tir_ref,
                  wen_ref, ben_ref, wee_ref, bee_ref,
                  wsd_ref, wer_ref, beb_ref, wup_ref, bup_ref,
                  wn1_ref, bn1_ref, wn2_ref, bn2_ref,
                  we1_ref, be1_ref, we2_ref, be2_ref,
                  out_ref, *, pairs):
    f32 = jnp.float32

    x = _mm(nf_ref[...], wen_ref[...]) + ben_ref[...]
    eenc = _mm(ef_ref[...], wee_ref[...]) + bee_ref[...]
    eb = _mm(eenc, wer_ref[...]) + beb_ref[...]

    fic = jnp.bitwise_and(fic_ref[0], _PAIR_N - 1)
    tic = jnp.bitwise_and(tic_ref[0], _PAIR_N - 1)
    fir = jnp.bitwise_and(fir_ref[0], _PAIR_N - 1)
    tir = jnp.bitwise_and(tir_ref[0], _PAIR_N - 1)

    lane = jax.lax.broadcasted_iota(jnp.int32, (_PAIR_E, _PAIR_N), 1)
    sub = jax.lax.broadcasted_iota(jnp.int32, (_PAIR_N, _PAIR_E), 0)

    ohf, oht, ohtf, ohtt = [], [], [], []
    for j in range(pairs):
        e0 = j * _PAIR_E
        fc = fic[e0:e0 + _PAIR_E, :]
        tc = tic[e0:e0 + _PAIR_E, :]
        fr = fir[:, e0:e0 + _PAIR_E]
        tr = tir[:, e0:e0 + _PAIR_E]
        ohf.append(jnp.where(fc == lane, 1.0, 0.0).astype(f32))
        oht.append(jnp.where(tc == lane, 1.0, 0.0).astype(f32))
        ohtf.append(jnp.where(fr == sub, 1.0, 0.0).astype(f32))
        ohtt.append(jnp.where(tr == sub, 1.0, 0.0).astype(f32))

    def messages(xcur):
        sds = []
        for j in range(pairs):
            xj = xcur[j * _PAIR_N:(j + 1) * _PAIR_N, :]
            src = _mm(ohf[j], xj)
            dst = _mm(oht[j], xj)
            sds.append(jnp.concatenate([src, dst], axis=1))
        sd = jnp.concatenate(sds, axis=0)
        return jax.nn.relu(_mm(sd, wsd_ref[...]) + eb)

    for _ in range(_N_PROP):
        msg = messages(x)
        aggs = []
        for j in range(pairs):
            mj = msg[j * _PAIR_E:(j + 1) * _PAIR_E, :]
            fwd = mj[:, :128]
            rev = mj[:, 128:]
            aggs.append(_mm(ohtt[j], fwd) + _mm(ohtf[j], rev))
        agg = jnp.concatenate(aggs, axis=0)
        xin = jnp.concatenate([x, agg], axis=1)
        x = _mm(xin, wup_ref[...]) + bup_ref[...]

    msg = messages(x)
    emsg = msg[:, :128] + msg[:, 128:]

    hn = _mm(jax.nn.relu(_mm(x, wn1_ref[...]) + bn1_ref[...]),
             wn2_ref[...]) + bn2_ref[...]
    he = _mm(jax.nn.relu(_mm(emsg, we1_ref[...]) + be1_ref[...]),
             we2_ref[...]) + be2_ref[...]

    sims_n, sims_e = [], []
    for j in range(pairs):
        n0 = j * _PAIR_N
        tq = hn[n0:n0 + _NPG, :]
        tc = hn[n0 + _NPG:n0 + _PAIR_N, :]
        sims_n.append(_mm_nt(tq, tc))
        e0 = j * _PAIR_E
        tqe = he[e0:e0 + _EPG, :]
        tce = he[e0 + _EPG:e0 + _PAIR_E, :]
        sims_e.append(_mm_nt(tqe, tce))

    pn_all, pe_all = _sinkhorn2(jnp.concatenate(sims_n, axis=0),
                                jnp.concatenate(sims_e, axis=0), pairs)

    scores = []
    for j in range(pairs):
        n0 = j * _PAIR_N
        tq = hn[n0:n0 + _NPG, :]
        tc = hn[n0 + _NPG:n0 + _PAIR_N, :]
        pn = pn_all[j * _NPG:(j + 1) * _NPG, :]
        s1 = jnp.sum(jax.nn.relu(tq - _mm(pn, tc)),
                     axis=(0, 1), keepdims=True)
        e0 = j * _PAIR_E
        tqe = he[e0:e0 + _EPG, :]
        tce = he[e0 + _EPG:e0 + _PAIR_E, :]
        pe = pe_all[j * _EPG:(j + 1) * _EPG, :]
        s2 = jnp.sum(jax.nn.relu(tqe - _mm(pe, tce)),
                     axis=(0, 1), keepdims=True)
        scores.append(-(s1 + s2))
    out_ref[...] = jnp.concatenate(scores, axis=1).reshape(1, 1, pairs)


def kernel(node_features, edge_features, from_idx, to_idx,
           W_enc_n, b_enc_n, W_enc_e, b_enc_e,
           W_msg, b_msg, W_rmsg, b_rmsg, W_upd, b_upd,
           Wn1, bn1, Wn2, bn2, We1, be1, We2, be2):
    f32 = jnp.float32
    n_total = node_features.shape[0]
    n_pairs = n_total // _PAIR_N
    e_total = n_pairs * _PAIR_E
    p = PAIRS_PER_BLOCK
    while n_pairs % p:
        p //= 2
    nb = n_pairs // p

    wm_s, wm_d, wm_e = W_msg[:64], W_msg[64:128], W_msg[128:]
    wr_d, wr_s, wr_e = W_rmsg[:64], W_rmsg[64:128], W_rmsg[128:]
    w_sd = jnp.concatenate([
        jnp.concatenate([wm_s, wr_s], axis=1),
        jnp.concatenate([wm_d, wr_d], axis=1),
    ], axis=0)
    w_er = jnp.concatenate([wm_e, wr_e], axis=1)
    b_eb = jnp.concatenate([b_msg, b_rmsg]).reshape(1, 256)

    fic = from_idx.reshape(nb, p * _PAIR_E, 1)
    tic = to_idx.reshape(nb, p * _PAIR_E, 1)
    fir = from_idx.reshape(nb, 1, p * _PAIR_E)
    tir = to_idx.reshape(nb, 1, p * _PAIR_E)

    def full(arr):
        a = jnp.asarray(arr, f32)
        if a.ndim == 1:
            a = a.reshape(1, -1)
        return a, pl.BlockSpec(a.shape, lambda i: (0,) * a.ndim)

    weights = [W_enc_n, b_enc_n, W_enc_e, b_enc_e,
               w_sd, w_er, b_eb, W_upd, b_upd,
               Wn1, bn1, Wn2, bn2, We1, be1, We2, be2]
    warrs, wspecs = zip(*[full(w) for w in weights])

    out = pl.pallas_call(
        functools.partial(_block_kernel, pairs=p),
        grid=(nb,),
        in_specs=[
            pl.BlockSpec((p * _PAIR_N, node_features.shape[1]),
                         lambda i: (i, 0)),
            pl.BlockSpec((p * _PAIR_E, edge_features.shape[1]),
                         lambda i: (i, 0)),
            pl.BlockSpec((1, p * _PAIR_E, 1), lambda i: (i, 0, 0)),
            pl.BlockSpec((1, p * _PAIR_E, 1), lambda i: (i, 0, 0)),
            pl.BlockSpec((1, 1, p * _PAIR_E), lambda i: (i, 0, 0)),
            pl.BlockSpec((1, 1, p * _PAIR_E), lambda i: (i, 0, 0)),
            *wspecs,
        ],
        out_specs=pl.BlockSpec((1, 1, p), lambda i: (i, 0, 0)),
        out_shape=jax.ShapeDtypeStruct((nb, 1, p), f32),
        compiler_params=pltpu.CompilerParams(
            dimension_semantics=(pltpu.GridDimensionSemantics.ARBITRARY,)),
    )(node_features, edge_features, fic, tic, fir, tir, *warrs)
    return out.reshape(n_pairs)
